# baseline (speedup 1.0000x reference)
import functools

import jax
import jax.numpy as jnp
from jax import lax
from jax.experimental import pallas as pl
from jax.experimental.pallas import tpu as pltpu
from jax.experimental.pallas import tpu_sc as plsc

_BATCH = 16384
_EMB = 64
_CHUNK = 128


def _sc_gather(table, idx2):
    info = plsc.get_sparse_core_info()
    nc, ns = info.num_cores, info.num_subcores
    nw = nc * ns
    n = idx2.shape[0]
    per_w = n // nw
    n_ch = per_w // _CHUNK
    mesh = plsc.VectorSubcoreMesh(core_axis_name="c", subcore_axis_name="s")

    @functools.partial(
        pl.kernel,
        mesh=mesh,
        compiler_params=pltpu.CompilerParams(use_tc_tiling_on_sc=False),
        out_type=jax.ShapeDtypeStruct((n, _EMB), jnp.float32),
        scratch_types=[
            pltpu.VMEM((per_w,), jnp.int32),
            pltpu.VMEM((_CHUNK, _EMB), jnp.float32),
            pltpu.VMEM((_CHUNK, _EMB), jnp.float32),
            pltpu.SemaphoreType.DMA,
            pltpu.SemaphoreType.DMA,
        ],
    )
    def k(idx_hbm, table_hbm, out_hbm, idx_v, rows_a, rows_b, sem_a, sem_b):
        wid = lax.axis_index("s") * nc + lax.axis_index("c")
        base = wid * per_w
        pltpu.sync_copy(idx_hbm.at[pl.ds(base, per_w)], idx_v)
        bufs = ((rows_a, sem_a), (rows_b, sem_b))
        copies = [None, None]
        for c in range(n_ch + 1):
            slot = c % 2
            if c < n_ch:
                rows_v, sem = bufs[slot]
                copies[slot] = pltpu.async_copy(
                    table_hbm.at[idx_v.at[pl.ds(c * _CHUNK, _CHUNK)]],
                    rows_v, sem)
            if c > 0:
                pslot = (c - 1) % 2
                copies[pslot].wait()
                pltpu.sync_copy(
                    bufs[pslot][0],
                    out_hbm.at[pl.ds(base + (c - 1) * _CHUNK, _CHUNK)])

    return k(idx2, table)


def _tc_mlp(xy, w1, b1, w2, b2, w3, b3):
    blk = 512
    n_blk = _BATCH // blk

    def body(x_ref, y_ref, w1_ref, b1_ref, w2_ref, b2_ref, w3_ref, b3_ref,
             o_ref):
        shared = x_ref[...] * y_ref[...]
        h1 = jnp.maximum(
            jnp.dot(shared, w1_ref[...], preferred_element_type=jnp.float32)
            + b1_ref[...], 0.0)
        h2 = jnp.maximum(
            jnp.dot(h1, w2_ref[...], preferred_element_type=jnp.float32)
            + b2_ref[...], 0.0)
        z = jnp.dot(h2, w3_ref[...], preferred_element_type=jnp.float32) \
            + b3_ref[...]
        o_ref[...] = jax.nn.sigmoid(z)

    zero2 = lambda i: (0, 0)
    return pl.pallas_call(
        body,
        grid=(n_blk,),
        in_specs=[
            pl.BlockSpec((blk, _EMB), lambda i: (i, 0)),
            pl.BlockSpec((blk, _EMB), lambda i: (i + n_blk, 0)),
            pl.BlockSpec((_EMB, 64), zero2),
            pl.BlockSpec((1, 64), zero2),
            pl.BlockSpec((64, 32), zero2),
            pl.BlockSpec((1, 32), zero2),
            pl.BlockSpec((32, 1), zero2),
            pl.BlockSpec((1, 1), zero2),
        ],
        out_specs=pl.BlockSpec((blk, 1), lambda i: (i, 0)),
        out_shape=jax.ShapeDtypeStruct((_BATCH, 1), jnp.float32),
    )(xy, xy, w1, b1, w2, b2, w3, b3)


def kernel(target_word, context_word, table, W1, b1, W2, b2, W3, b3):
    idx2 = jnp.concatenate([target_word, context_word]).astype(jnp.int32)
    xy = _sc_gather(table, idx2)
    out = _tc_mlp(xy, W1, b1.reshape(1, -1), W2, b2.reshape(1, -1), W3,
                  b3.reshape(1, 1))
    return jnp.reshape(out, (-1,))


# speedup vs baseline: 1.0381x; 1.0381x over previous
import functools

import jax
import jax.numpy as jnp
from jax import lax
from jax.experimental import pallas as pl
from jax.experimental.pallas import tpu as pltpu
from jax.experimental.pallas import tpu_sc as plsc

_BATCH = 16384
_EMB = 64
_CHUNK = 128


def _sc_gather(table, tgt, ctx):
    info = plsc.get_sparse_core_info()
    nc, ns = info.num_cores, info.num_subcores
    nw = nc * ns
    b = tgt.shape[0]
    per_w = b // nw
    n_ch = per_w // _CHUNK
    mesh = plsc.VectorSubcoreMesh(core_axis_name="c", subcore_axis_name="s")

    @functools.partial(
        pl.kernel,
        mesh=mesh,
        compiler_params=pltpu.CompilerParams(use_tc_tiling_on_sc=False),
        out_type=jax.ShapeDtypeStruct((b, 2 * _EMB), jnp.float32),
        scratch_types=[
            pltpu.VMEM((per_w,), jnp.int32),
            pltpu.VMEM((per_w,), jnp.int32),
            pltpu.VMEM((_CHUNK, _EMB), jnp.float32),
            pltpu.VMEM((_CHUNK, _EMB), jnp.float32),
            pltpu.VMEM((_CHUNK, _EMB), jnp.float32),
            pltpu.VMEM((_CHUNK, _EMB), jnp.float32),
            pltpu.SemaphoreType.DMA,
            pltpu.SemaphoreType.DMA,
            pltpu.SemaphoreType.DMA,
            pltpu.SemaphoreType.DMA,
        ],
    )
    def k(tgt_hbm, ctx_hbm, table_hbm, out_hbm, ti_v, ci_v,
          rows_a, rows_b, rows_c, rows_d, s_a, s_b, s_c, s_d):
        wid = lax.axis_index("s") * nc + lax.axis_index("c")
        base = wid * per_w
        pltpu.sync_copy(tgt_hbm.at[pl.ds(base, per_w)], ti_v)
        pltpu.sync_copy(ctx_hbm.at[pl.ds(base, per_w)], ci_v)
        bufs = ((rows_a, s_a), (rows_b, s_b), (rows_c, s_c), (rows_d, s_d))
        copies = [None] * 4
        n_items = 2 * n_ch
        for c in range(n_items + 2):
            if c < n_items:
                ch, side = c // 2, c % 2
                idx_ref = (ti_v, ci_v)[side]
                rows, sem = bufs[c % 4]
                copies[c % 4] = pltpu.async_copy(
                    table_hbm.at[idx_ref.at[pl.ds(ch * _CHUNK, _CHUNK)]],
                    rows, sem)
            if c >= 2:
                p = c - 2
                ch, side = p // 2, p % 2
                rows, _ = bufs[p % 4]
                copies[p % 4].wait()
                pltpu.sync_copy(
                    rows,
                    out_hbm.at[pl.ds(base + ch * _CHUNK, _CHUNK),
                               pl.ds(side * _EMB, _EMB)])

    return k(tgt, ctx, table)


def _tc_mlp(xy, w1, b1, w2, b2, w3, b3):
    blk = 1024
    n_blk = _BATCH // blk

    def body(xy_ref, w1_ref, b1_ref, w2_ref, b2_ref, w3_ref, b3_ref, o_ref):
        shared = xy_ref[:, :_EMB] * xy_ref[:, _EMB:]
        h1 = jnp.maximum(
            jnp.dot(shared, w1_ref[...], preferred_element_type=jnp.float32)
            + b1_ref[...], 0.0)
        h2 = jnp.maximum(
            jnp.dot(h1, w2_ref[...], preferred_element_type=jnp.float32)
            + b2_ref[...], 0.0)
        z = jnp.dot(h2, w3_ref[...], preferred_element_type=jnp.float32) \
            + b3_ref[...]
        o_ref[...] = jax.nn.sigmoid(z)

    zero2 = lambda i: (0, 0)
    return pl.pallas_call(
        body,
        grid=(n_blk,),
        in_specs=[
            pl.BlockSpec((blk, 2 * _EMB), lambda i: (i, 0)),
            pl.BlockSpec((_EMB, 64), zero2),
            pl.BlockSpec((1, 64), zero2),
            pl.BlockSpec((64, 32), zero2),
            pl.BlockSpec((1, 32), zero2),
            pl.BlockSpec((32, 1), zero2),
            pl.BlockSpec((1, 1), zero2),
        ],
        out_specs=pl.BlockSpec((blk, 1), lambda i: (i, 0)),
        out_shape=jax.ShapeDtypeStruct((_BATCH, 1), jnp.float32),
    )(xy, w1, b1, w2, b2, w3, b3)


def kernel(target_word, context_word, table, W1, b1, W2, b2, W3, b3):
    xy = _sc_gather(table, target_word.astype(jnp.int32),
                    context_word.astype(jnp.int32))
    out = _tc_mlp(xy, W1, b1.reshape(1, -1), W2, b2.reshape(1, -1), W3,
                  b3.reshape(1, 1))
    return jnp.reshape(out, (-1,))
